# baseline (speedup 1.0000x reference)
import functools

import jax
import jax.numpy as jnp
from jax import lax
from jax.experimental import pallas as pl
from jax.experimental.pallas import tpu as pltpu
from jax.experimental.pallas import tpu_sc as plsc

_NC = 2
_NS = 16
_NW = _NC * _NS
_L = 16


@functools.cache
def _build(S, T, D):
    B = S * T
    b_per_w = B // _NW
    SB = S // _NW
    n_pairs = T // 2
    n_jg = SB // _L
    mesh = plsc.VectorSubcoreMesh(core_axis_name="c", subcore_axis_name="s")

    @functools.partial(
        pl.kernel,
        out_type=jax.ShapeDtypeStruct((T, D, S), jnp.float32),
        mesh=mesh,
        scratch_types=[
            pltpu.VMEM((b_per_w,), jnp.int32),
            pltpu.VMEM((SB,), jnp.int32),
            pltpu.VMEM((SB,), jnp.int32),
            pltpu.VMEM((SB,), jnp.int32),
            pltpu.VMEM((SB,), jnp.int32),
            pltpu.VMEM((SB, 2 * D), jnp.float32),
            pltpu.VMEM((SB, 2 * D), jnp.float32),
            pltpu.VMEM((D, SB), jnp.float32),
            pltpu.VMEM((D, SB), jnp.float32),
            pltpu.SemaphoreType.DMA,
            pltpu.SemaphoreType.DMA,
            pltpu.SemaphoreType.DMA,
            pltpu.SemaphoreType.DMA,
        ],
        compiler_params=pltpu.CompilerParams(
            use_tc_tiling_on_sc=True, needs_layout_passes=False
        ),
    )
    def gather_kernel(t2_hbm, ev_hbm, out_hbm,
                      slab, qA, qB, pA, pB, GA, GB, tbA, tbB,
                      gsA, gsB, wsA, wsB):
        wid = lax.axis_index("s") * _NC + lax.axis_index("c")
        base = pl.multiple_of(wid * b_per_w, 8)
        s0 = pl.multiple_of(wid * SB, SB)
        pltpu.sync_copy(ev_hbm.at[pl.ds(base, b_per_w)], slab)

        iota = lax.iota(jnp.int32, _L)
        iota_t = iota * T

        def prep(t, qb, pb):
            for j in range(n_jg):
                vec = iota_t + (j * _L * T + t)
                r = plsc.load_gather(slab, [vec])
                qb[pl.ds(j * _L, _L)] = jnp.right_shift(r, 1)
                pb[pl.ds(j * _L, _L)] = jnp.left_shift(jnp.bitwise_and(r, 1), 6)

        def fire_g(qb, G, sem):
            pltpu.async_copy(t2_hbm.at[qb], G, sem)

        def wait_g(G, sem):
            pltpu.make_async_copy(t2_hbm.at[pl.ds(0, SB), :], G, sem).wait()

        def wait_w(tb, sem):
            pltpu.make_async_copy(tb, out_hbm.at[0, :, pl.ds(0, SB)], sem).wait()

        def transpose_write(t, pb, G, tb, wsem):
            for j in range(n_jg):
                pv = pb[pl.ds(j * _L, _L)]
                rv = iota + (j * _L)
                for d in range(D):
                    cv = pv + d
                    tb[d, pl.ds(j * _L, _L)] = plsc.load_gather(G, [rv, cv])
            pltpu.async_copy(tb, out_hbm.at[t, :, pl.ds(s0, SB)], wsem)

        prep(0, qA, pA)
        fire_g(qA, GA, gsA)

        def body(i, carry):
            t0 = 2 * i
            prep(t0 + 1, qB, pB)
            fire_g(qB, GB, gsB)
            wait_g(GA, gsA)

            @pl.when(i > 0)
            def _wA():
                wait_w(tbA, wsA)

            transpose_write(t0, pA, GA, tbA, wsA)

            @pl.when(i < n_pairs - 1)
            def _nextA():
                prep(t0 + 2, qA, pA)
                fire_g(qA, GA, gsA)

            wait_g(GB, gsB)

            @pl.when(i > 0)
            def _wB():
                wait_w(tbB, wsB)

            transpose_write(t0 + 1, pB, GB, tbB, wsB)
            return carry

        lax.fori_loop(0, n_pairs, body, 0)
        wait_w(tbA, wsA)
        wait_w(tbB, wsB)

    return gather_kernel


def kernel(event, table):
    S, T = event.shape
    D = table.shape[1]
    flat = event.reshape(S * T)
    t2 = jnp.concatenate(
        [table, jnp.zeros((1, D), table.dtype)], axis=0
    ).reshape(-1, 2 * D)
    P = _build(S, T, D)(t2, flat)
    return jnp.transpose(P, (2, 0, 1))


# speedup vs baseline: 2.0054x; 2.0054x over previous
import functools

import jax
import jax.numpy as jnp
from jax import lax
from jax.experimental import pallas as pl
from jax.experimental.pallas import tpu as pltpu
from jax.experimental.pallas import tpu_sc as plsc

_NC = 2
_NS = 16
_NW = _NC * _NS
_L = 16


@functools.cache
def _build(S, T, D):
    B = S * T
    b_per_w = B // _NW
    SB = S // _NW
    n_pairs = T // 2
    n_jg = SB // _L
    mesh = plsc.VectorSubcoreMesh(core_axis_name="c", subcore_axis_name="s")

    @functools.partial(
        pl.kernel,
        out_type=jax.ShapeDtypeStruct((T, D, S), jnp.float32),
        mesh=mesh,
        scratch_types=[
            pltpu.VMEM((b_per_w,), jnp.int32),
            pltpu.VMEM((SB,), jnp.int32),
            pltpu.VMEM((SB,), jnp.int32),
            pltpu.VMEM((SB, 2 * D), jnp.float32),
            pltpu.VMEM((SB, 2 * D), jnp.float32),
            pltpu.VMEM((D, SB), jnp.float32),
            pltpu.VMEM((D, SB), jnp.float32),
            pltpu.SemaphoreType.DMA,
            pltpu.SemaphoreType.DMA,
            pltpu.SemaphoreType.DMA,
            pltpu.SemaphoreType.DMA,
        ],
        compiler_params=pltpu.CompilerParams(
            use_tc_tiling_on_sc=True, needs_layout_passes=False
        ),
    )
    def gather_kernel(t3_hbm, ev_hbm, out_hbm,
                      slab, qA, qB, GA, GB, tbA, tbB,
                      gsA, gsB, wsA, wsB):
        wid = lax.axis_index("s") * _NC + lax.axis_index("c")
        base = pl.multiple_of(wid * b_per_w, 8)
        s0 = pl.multiple_of(wid * SB, SB)
        pltpu.sync_copy(ev_hbm.at[pl.ds(base, b_per_w)], slab)

        iota = lax.iota(jnp.int32, _L)
        iota_t = iota * T

        def prep(t, qb):
            for j in range(n_jg):
                vec = iota_t + (j * _L * T + t)
                qb[pl.ds(j * _L, _L)] = plsc.load_gather(slab, [vec])

        def fire_g(qb, G, sem):
            pltpu.async_copy(t3_hbm.at[qb], G, sem)

        def wait_g(G, sem):
            pltpu.make_async_copy(t3_hbm.at[pl.ds(0, SB), :], G, sem).wait()

        def wait_w(tb, sem):
            pltpu.make_async_copy(tb, out_hbm.at[0, :, pl.ds(0, SB)], sem).wait()

        def transpose_write(t, G, tb, wsem):
            @plsc.parallel_loop(0, D, 1, unroll=8)
            def _tp(d):
                dv = jnp.broadcast_to(d, (_L,))
                for j in range(n_jg):
                    rv = iota + (j * _L)
                    tb[d, pl.ds(j * _L, _L)] = plsc.load_gather(G, [rv, dv])

            pltpu.async_copy(tb, out_hbm.at[t, :, pl.ds(s0, SB)], wsem)

        prep(0, qA)
        fire_g(qA, GA, gsA)

        def body(i, carry):
            t0 = 2 * i
            prep(t0 + 1, qB)
            fire_g(qB, GB, gsB)
            wait_g(GA, gsA)

            @pl.when(i > 0)
            def _wA():
                wait_w(tbA, wsA)

            transpose_write(t0, GA, tbA, wsA)

            @pl.when(i < n_pairs - 1)
            def _nextA():
                prep(t0 + 2, qA)
                fire_g(qA, GA, gsA)

            wait_g(GB, gsB)

            @pl.when(i > 0)
            def _wB():
                wait_w(tbB, wsB)

            transpose_write(t0 + 1, GB, tbB, wsB)
            return carry

        lax.fori_loop(0, n_pairs, body, 0)
        wait_w(tbA, wsA)
        wait_w(tbB, wsB)

    return gather_kernel


def kernel(event, table):
    S, T = event.shape
    D = table.shape[1]
    flat = event.reshape(S * T)
    t3 = jnp.pad(table, ((0, 0), (0, D)))
    P = _build(S, T, D)(t3, flat)
    return jnp.transpose(P, (2, 0, 1))
